# baseline (speedup 1.0000x reference)
import functools

import jax
import jax.numpy as jnp
from jax import lax
from jax.experimental import pallas as pl
from jax.experimental.pallas import tpu as pltpu
from jax.experimental.pallas import tpu_sc as plsc

N = 10000
D = 128
NC = 2
NS = 16
NW = NC * NS
NPAD = 10112
RPT = NPAD // NS
KC = 48
SB = 8
NBUF = 6
BSUP = 3
BCH = BSUP * SB

_mesh = plsc.VectorSubcoreMesh(
    core_axis_name="c", subcore_axis_name="s", num_cores=NC, num_subcores=NS
)


def _hist_body(nsup, dst_hbm, ones_hbm, z8_hbm, degp_hbm, dst_sb, ones_v, acc8):
    c = lax.axis_index("c")
    s = lax.axis_index("s")
    wid = s * NC + c
    r0 = s * RPT
    pltpu.sync_copy(z8_hbm.at[pl.ds(r0, RPT)], acc8.at[pl.ds(r0, RPT)])
    pltpu.sync_copy(ones_hbm, ones_v)
    plsc.subcore_barrier()
    sup0 = wid * nsup

    def body(g, _):
        pltpu.sync_copy(dst_hbm.at[sup0 + g], dst_sb)
        for j in range(SB):
            pltpu.sync_copy(ones_v, acc8.at[dst_sb.at[j]], add=True)
        return ()

    lax.fori_loop(0, nsup, body, ())
    plsc.subcore_barrier()
    pltpu.sync_copy(acc8.at[pl.ds(r0, RPT)], degp_hbm.at[c, pl.ds(r0, RPT)])


def _agg_body(nsup, src_hbm, dst_hbm, y_hbm, z_hbm, out_hbm,
              src_sb, dst_sb, gb0, gb1, gb2, gb3, gb4, gb5, acc,
              s0, s1, s2, s3, s4, s5):
    c = lax.axis_index("c")
    s = lax.axis_index("s")
    wid = s * NC + c
    r0 = s * RPT
    pltpu.sync_copy(z_hbm.at[pl.ds(r0, RPT)], acc.at[pl.ds(r0, RPT)])
    plsc.subcore_barrier()
    sup0 = wid * nsup
    nblk = nsup // BSUP

    gbufs = (gb0, gb1, gb2, gb3, gb4, gb5)
    sems = (s0, s1, s2, s3, s4, s5)

    def g_start(u, j, b):
        pltpu.make_async_copy(
            y_hbm.at[src_sb.at[u, j]], gbufs[b], sems[b]).start()

    def g_wait(u, j, b):
        pltpu.make_async_copy(
            y_hbm.at[src_sb.at[u, j]], gbufs[b], sems[b]).wait()

    def stage(gsup, u):
        pltpu.sync_copy(src_hbm.at[gsup], src_sb.at[u])
        pltpu.sync_copy(dst_hbm.at[gsup], dst_sb.at[u])

    def block(k, last):
        for u in range(BSUP):
            if not last:
                stage(sup0 + k * BSUP + u + 1, (u + 1) % BSUP)
            elif u + 1 < BSUP:
                stage(sup0 + k * BSUP + u + 1, (u + 1) % BSUP)
            for j in range(SB):
                i = u * SB + j
                g_wait(u, j, i % NBUF)
                pltpu.sync_copy(gbufs[i % NBUF], acc.at[dst_sb.at[u, j]],
                                add=True)
                i2 = i + NBUF - 2
                if i2 < BCH:
                    g_start(i2 // SB, i2 % SB, i2 % NBUF)
                elif not last:
                    g_start(0, i2 % SB, i2 % NBUF)

    stage(sup0, 0)
    for t in range(NBUF - 2):
        g_start(0, t, t)

    lax.fori_loop(0, nblk - 1, lambda k, _: (block(k, False), ())[1], ())
    block(nblk - 1, True)

    plsc.subcore_barrier()
    pltpu.sync_copy(acc.at[pl.ds(r0, RPT)], out_hbm.at[c, pl.ds(r0, RPT)])


def _mm_body(xp_ref, w_ref, h0_ref, h1_ref, y_ref, dinvb_ref):
    deg = h0_ref[:, 0:1] + h1_ref[:, 0:1] + 1.0
    dinv = 1.0 / jnp.sqrt(deg)
    xw = jnp.dot(xp_ref[...], w_ref[...], preferred_element_type=jnp.float32)
    y_ref[...] = xw * dinv
    dinvb_ref[...] = jnp.broadcast_to(dinv, (NPAD, D))


def _fin_body(p0_ref, p1_ref, y_ref, dinvb_ref, b_ref, a_ref, o_ref):
    h = dinvb_ref[...] * (p0_ref[...] + p1_ref[...] + y_ref[...]) + b_ref[...]
    res = jnp.where(h > 0, h, a_ref[...] * h)
    o_ref[...] = res[:N, :]


def kernel(x, edge_index, W, b, prelu_a):
    E = edge_index.shape[1]
    totch_min = -(-E // KC)
    nck = -(-totch_min // (NW * BCH)) * BCH
    totch = NW * nck
    nsup = nck // SB
    epad = totch * KC

    pad = jnp.full((epad - E,), N, dtype=jnp.int32)
    src_r = jnp.concatenate([edge_index[0], pad]).reshape(totch // SB, SB, KC)
    dst_r = jnp.concatenate([edge_index[1], pad]).reshape(totch // SB, SB, KC)

    xp = jnp.pad(x, ((0, NPAD - N), (0, 0)))
    ones_row = jnp.zeros((KC, 8), jnp.float32).at[:, 0].set(1.0)
    z8 = jnp.zeros((NPAD, 8), jnp.float32)
    z128 = jnp.zeros((NPAD, D), jnp.float32)

    hist = pl.kernel(
        functools.partial(_hist_body, nsup),
        out_type=jax.ShapeDtypeStruct((NC, NPAD, 8), jnp.float32),
        mesh=_mesh,
        scratch_types=[
            pltpu.VMEM((SB, KC), jnp.int32),
            pltpu.VMEM((KC, 8), jnp.float32),
            pltpu.VMEM_SHARED((NPAD, 8), jnp.float32),
        ],
    )
    degp = hist(dst_r, ones_row, z8)

    y, dinvb = pl.pallas_call(
        _mm_body,
        out_shape=[
            jax.ShapeDtypeStruct((NPAD, D), jnp.float32),
            jax.ShapeDtypeStruct((NPAD, D), jnp.float32),
        ],
    )(xp, W, degp[0], degp[1])

    agg = pl.kernel(
        functools.partial(_agg_body, nsup),
        out_type=jax.ShapeDtypeStruct((NC, NPAD, D), jnp.float32),
        mesh=_mesh,
        scratch_types=(
            [pltpu.VMEM((BSUP, SB, KC), jnp.int32)] * 2
            + [pltpu.VMEM((KC, D), jnp.float32)] * NBUF
            + [pltpu.VMEM_SHARED((NPAD, D), jnp.float32)]
            + [pltpu.SemaphoreType.DMA] * NBUF
        ),
    )(src_r, dst_r, y, z128)

    b2 = jnp.broadcast_to(b.reshape(1, D), (1, D))
    a2 = jnp.broadcast_to(prelu_a.reshape(1, 1), (1, D))
    out = pl.pallas_call(
        _fin_body,
        out_shape=jax.ShapeDtypeStruct((N, D), jnp.float32),
    )(agg[0], agg[1], y, dinvb, b2, a2)
    return out


# speedup vs baseline: 1.2422x; 1.2422x over previous
import functools

import jax
import jax.numpy as jnp
from jax import lax
from jax.experimental import pallas as pl
from jax.experimental.pallas import tpu as pltpu
from jax.experimental.pallas import tpu_sc as plsc

N = 10000
D = 128
NS = 16
NPAD = 10112
RPT = NPAD // NS
KC = 128
SB = 8

_mesh = plsc.VectorSubcoreMesh(
    core_axis_name="c", subcore_axis_name="s", num_cores=1, num_subcores=NS
)


def _hist_body(nsup, dst_hbm, ones_hbm, z8_hbm, degp_hbm, dst_sb, ones_v, acc8):
    s = lax.axis_index("s")
    r0 = s * RPT
    pltpu.sync_copy(z8_hbm.at[pl.ds(r0, RPT)], acc8.at[pl.ds(r0, RPT)])
    pltpu.sync_copy(ones_hbm, ones_v)
    plsc.subcore_barrier()
    sup0 = s * nsup

    def body(g, _):
        pltpu.sync_copy(dst_hbm.at[sup0 + g], dst_sb)
        for j in range(SB):
            pltpu.sync_copy(ones_v, acc8.at[dst_sb.at[j]], add=True)
        return ()

    lax.fori_loop(0, nsup, body, ())
    plsc.subcore_barrier()
    pltpu.sync_copy(acc8.at[pl.ds(r0, RPT)], degp_hbm.at[pl.ds(r0, RPT)])


def _agg_body(nsup, src_hbm, dst_hbm, y_hbm, z_hbm, out_hbm,
              src_sb, dst_sb, gb0, gb1, acc, semA, semB):
    s = lax.axis_index("s")
    r0 = s * RPT
    pltpu.sync_copy(z_hbm.at[pl.ds(r0, RPT)], acc.at[pl.ds(r0, RPT)])
    plsc.subcore_barrier()
    sup0 = s * nsup

    gbufs = (gb0, gb1)
    sems = (semA, semB)

    def body(g, _):
        pltpu.sync_copy(src_hbm.at[sup0 + g], src_sb)
        pltpu.sync_copy(dst_hbm.at[sup0 + g], dst_sb)
        pltpu.make_async_copy(y_hbm.at[src_sb.at[0]], gb0, semA).start()
        pltpu.make_async_copy(y_hbm.at[src_sb.at[1]], gb1, semB).start()
        for j in range(SB):
            gb, sem = gbufs[j % 2], sems[j % 2]
            pltpu.make_async_copy(y_hbm.at[src_sb.at[j]], gb, sem).wait()
            pltpu.sync_copy(gb, acc.at[dst_sb.at[j]], add=True)
            if j + 2 < SB:
                pltpu.make_async_copy(
                    y_hbm.at[src_sb.at[j + 2]], gb, sem).start()
        return ()

    lax.fori_loop(0, nsup, body, ())
    plsc.subcore_barrier()
    pltpu.sync_copy(acc.at[pl.ds(r0, RPT)], out_hbm.at[pl.ds(r0, RPT)])


def _mm_body(xp_ref, w_ref, h_ref, y_ref, dinvb_ref):
    deg = h_ref[:, 0:1] + 1.0
    dinv = 1.0 / jnp.sqrt(deg)
    xw = jnp.dot(xp_ref[...], w_ref[...], preferred_element_type=jnp.float32)
    y_ref[...] = xw * dinv
    dinvb_ref[...] = jnp.broadcast_to(dinv, (NPAD, D))


def _fin_body(p_ref, y_ref, dinvb_ref, b_ref, a_ref, o_ref):
    h = dinvb_ref[...] * (p_ref[...] + y_ref[...]) + b_ref[...]
    res = jnp.where(h > 0, h, a_ref[...] * h)
    o_ref[...] = res[:N, :]


def kernel(x, edge_index, W, b, prelu_a):
    E = edge_index.shape[1]
    totch_min = -(-E // KC)
    nck = -(-totch_min // (NS * SB)) * SB
    totch = NS * nck
    nsup = nck // SB
    epad = totch * KC

    pad = jnp.full((epad - E,), N, dtype=jnp.int32)
    src_r = jnp.concatenate([edge_index[0], pad]).reshape(totch // SB, SB, KC)
    dst_r = jnp.concatenate([edge_index[1], pad]).reshape(totch // SB, SB, KC)

    xp = jnp.pad(x, ((0, NPAD - N), (0, 0)))
    ones_row = jnp.zeros((KC, 8), jnp.float32).at[:, 0].set(1.0)
    z8 = jnp.zeros((NPAD, 8), jnp.float32)
    z128 = jnp.zeros((NPAD, D), jnp.float32)

    hist = pl.kernel(
        functools.partial(_hist_body, nsup),
        out_type=jax.ShapeDtypeStruct((NPAD, 8), jnp.float32),
        mesh=_mesh,
        scratch_types=[
            pltpu.VMEM((SB, KC), jnp.int32),
            pltpu.VMEM((KC, 8), jnp.float32),
            pltpu.VMEM_SHARED((NPAD, 8), jnp.float32),
        ],
    )
    degp = hist(dst_r, ones_row, z8)

    y, dinvb = pl.pallas_call(
        _mm_body,
        out_shape=[
            jax.ShapeDtypeStruct((NPAD, D), jnp.float32),
            jax.ShapeDtypeStruct((NPAD, D), jnp.float32),
        ],
    )(xp, W, degp)

    agg = pl.kernel(
        functools.partial(_agg_body, nsup),
        out_type=jax.ShapeDtypeStruct((NPAD, D), jnp.float32),
        mesh=_mesh,
        scratch_types=[
            pltpu.VMEM((SB, KC), jnp.int32),
            pltpu.VMEM((SB, KC), jnp.int32),
            pltpu.VMEM((KC, D), jnp.float32),
            pltpu.VMEM((KC, D), jnp.float32),
            pltpu.VMEM_SHARED((NPAD, D), jnp.float32),
            pltpu.SemaphoreType.DMA,
            pltpu.SemaphoreType.DMA,
        ],
    )(src_r, dst_r, y, z128)

    b2 = jnp.broadcast_to(b.reshape(1, D), (1, D))
    a2 = jnp.broadcast_to(prelu_a.reshape(1, 1), (1, D))
    out = pl.pallas_call(
        _fin_body,
        out_shape=jax.ShapeDtypeStruct((N, D), jnp.float32),
    )(agg, y, dinvb, b2, a2)
    return out
